# baseline (speedup 1.0000x reference)
import functools

import jax
import jax.numpy as jnp
from jax import lax
from jax.experimental import pallas as pl
from jax.experimental.pallas import tpu as pltpu
from jax.experimental.pallas import tpu_sc as plsc

_N = 10000
_D = 128
_H = 128
_HALF = 64
_OUT = 2
_NEG = 0.01

_NPAD = 10112
_RB = 632
_NBLK = _NPAD // _RB
_NCORE = 2
_NSUB = 16
_NW = _NCORE * _NSUB
_CH = 128
_CPW = 80
_NCHUNK = _NW * _CPW
_EPAD = _NCHUNK * _CH
_CPT = _NCHUNK // _NSUB
_HD = _D // 2

_f32 = jnp.float32
_E = 320000

_vmesh = plsc.VectorSubcoreMesh(core_axis_name="core", subcore_axis_name="subcore")
_sc_params = pltpu.CompilerParams(needs_layout_passes=False)


def _hist_body(col_hbm, out_hbm, col_v, hist_v, blk_v, acc_v, h_sh):
    cid = lax.axis_index("core")
    sid = lax.axis_index("subcore")
    w = cid * _NSUB + sid
    pltpu.sync_copy(col_hbm.at[pl.ds(w * _CPW, _CPW)], col_v)

    @pl.loop(0, _NPAD // 16)
    def _zero(i):
        hist_v[pl.ds(i * 16, 16)] = jnp.zeros((16,), _f32)

    ones = jnp.ones((16,), _f32)

    @pl.loop(0, _CPW)
    def _chunk(j):
        @pl.loop(0, _CH // 16)
        def _vec(c):
            iv = col_v[j, pl.ds(c * 16, 16)]
            plsc.addupdate_scatter(hist_v, [iv], ones)

    pltpu.sync_copy(hist_v, h_sh.at[sid])
    plsc.subcore_barrier()

    _HB = _NPAD // _CH
    _BPS = 5

    @pl.loop(0, _BPS)
    def _rb(t):
        blk = sid * _BPS + t

        @pl.when(blk < _HB)
        def _one():
            off = pl.multiple_of(blk * _CH, _CH)
            pltpu.sync_copy(h_sh.at[:, pl.ds(off, _CH)], blk_v)

            @pl.loop(0, _CH // 16)
            def _acc(i):
                s = blk_v[0, pl.ds(i * 16, 16)]
                for j in range(1, _NSUB):
                    s = s + blk_v[j, pl.ds(i * 16, 16)]
                acc_v[pl.ds(i * 16, 16)] = s

            pltpu.sync_copy(acc_v, out_hbm.at[cid, pl.ds(off, _CH)])


@jax.jit
def _hist_call(col2):
    return pl.kernel(
        _hist_body,
        out_type=jax.ShapeDtypeStruct((_NCORE, _NPAD), _f32),
        mesh=_vmesh,
        compiler_params=_sc_params,
        scratch_types=[
            pltpu.VMEM((_CPW, _CH), jnp.int32),
            pltpu.VMEM((_NPAD,), _f32),
            pltpu.VMEM((_NSUB, _CH), _f32),
            pltpu.VMEM((_CH,), _f32),
            pltpu.VMEM_SHARED((_NSUB, _NPAD), _f32),
        ],
    )(col2)


_NBUF = 2
_MASK = (1 << 14) - 1


def _agg_body(p0_hbm, p1_hbm, pk_hbm, q_hbm, pk_v, ridx_r, cidx_r, rows_v,
              p_sh, q_sh, *sems):
    cid = lax.axis_index("core")
    sid = lax.axis_index("subcore")

    @pl.when(cid == 0)
    def _s0():
        pltpu.sync_copy(p0_hbm.at[pl.ds(sid * _RB, _RB)],
                        p_sh.at[pl.ds(sid * _RB, _RB)])

    @pl.when(cid == 1)
    def _s1():
        pltpu.sync_copy(p1_hbm.at[pl.ds(sid * _RB, _RB)],
                        p_sh.at[pl.ds(sid * _RB, _RB)])

    @pl.loop(0, _CH)
    def _zr(i):
        @pl.loop(0, _HD // 16)
        def _zc(c):
            rows_v[i, pl.ds(c * 16, 16)] = jnp.zeros((16,), _f32)

    @pl.loop(0, _RB // _CH)
    def _zs(k):
        pltpu.sync_copy(rows_v.at[pl.ds(0, _CH)],
                        q_sh.at[pl.ds(sid * _RB + k * _CH, _CH)])

    _tail = _RB - (_RB // _CH) * _CH
    pltpu.sync_copy(rows_v.at[pl.ds(0, _tail)],
                    q_sh.at[pl.ds(sid * _RB + (_RB // _CH) * _CH, _tail)])

    pltpu.sync_copy(pk_hbm.at[pl.ds(sid * _CPT, _CPT)], pk_v)

    plsc.subcore_barrier()

    def _buf(b):
        return rows_v.at[pl.ds(b * _CH, _CH)]

    def _unpack(c, b):
        for k in range(_CH // 16):
            pk = pk_v[c, pl.ds(k * 16, 16)]
            ridx_r[b, pl.ds(k * 16, 16)] = pk & _MASK
            cidx_r[b, pl.ds(k * 16, 16)] = pk >> 14

    def _gather(b):
        pltpu.async_copy(p_sh.at[ridx_r.at[b]], _buf(b), sems[b])

    for b in range(_NBUF):
        _unpack(b, b)
        _gather(b)

    @pl.loop(0, _CPT // _NBUF)
    def _steady(s):
        for b in range(_NBUF):
            c = s * _NBUF + b
            pltpu.make_async_copy(p_sh.at[ridx_r.at[b]], _buf(b),
                                  sems[b]).wait()
            pltpu.sync_copy(_buf(b), q_sh.at[cidx_r.at[b]], add=True)
            _unpack(lax.rem(c + _NBUF, _CPT), b)
            _gather(b)

    for b in range(_NBUF):
        pltpu.make_async_copy(p_sh.at[ridx_r.at[b]], _buf(b), sems[b]).wait()

    plsc.subcore_barrier()

    pltpu.sync_copy(q_sh.at[pl.ds(sid * _RB, _RB)],
                    q_hbm.at[cid, pl.ds(sid * _RB, _RB)])


@jax.jit
def _agg_call(p0, p1, packed):
    return pl.kernel(
        _agg_body,
        out_type=jax.ShapeDtypeStruct((_NCORE, _NPAD, _HD), _f32),
        mesh=_vmesh,
        compiler_params=pltpu.CompilerParams(use_tc_tiling_on_sc=False),
        scratch_types=[
            pltpu.VMEM((_CPT, _CH), jnp.int32),
            pltpu.VMEM((_NBUF, _CH), jnp.int32),
            pltpu.VMEM((_NBUF, _CH), jnp.int32),
            pltpu.VMEM((_NBUF * _CH, _HD), _f32),
            pltpu.VMEM_SHARED((_NPAD, _HD), _f32),
            pltpu.VMEM_SHARED((_NPAD, _HD), _f32),
        ] + [pltpu.SemaphoreType.DMA] * _NBUF,
    )(p0, p1, packed)


_EB = 8192
_EGRID = _EPAD // _EB


def _pack_body(ei_ref, pk_ref, col_ref):
    i = pl.program_id(0)
    base = i * _EB
    off = base + jax.lax.broadcasted_iota(jnp.int32, (_EB // _CH, _CH), 0) * _CH \
        + jax.lax.broadcasted_iota(jnp.int32, (_EB // _CH, _CH), 1)
    valid = off < _E
    row = jnp.where(valid, ei_ref[0], 0)
    col = jnp.where(valid, ei_ref[1], _NPAD - 1)
    pk_ref[...] = row | (col << 14)
    col_ref[...] = col


@jax.jit
def _pack_call(ei):
    return pl.pallas_call(
        _pack_body,
        grid=(_EGRID,),
        in_specs=[pl.BlockSpec((2, _EB // _CH, _CH), lambda i: (0, i, 0))],
        out_specs=[pl.BlockSpec((_EB // _CH, _CH), lambda i: (i, 0)),
                   pl.BlockSpec((_EB // _CH, _CH), lambda i: (i, 0))],
        out_shape=[jax.ShapeDtypeStruct((_NCHUNK, _CH), jnp.int32),
                   jax.ShapeDtypeStruct((_NCHUNK, _CH), jnp.int32)],
    )(ei)


def _leaky(v):
    return jnp.where(v > 0, v, v * _NEG)


def _dense1_body(x_ref, deg_ref, wi, bi, wn, bn, wa, ba, wt, bt, va,
                 p0_ref, p1_ref):
    xb = x_ref[...]
    h = _leaky(jnp.dot(xb, wi[...], preferred_element_type=_f32) + bi[...])
    xn = jnp.dot(h[:, :_HALF], wn[...], preferred_element_type=_f32) + bn[...]
    xa = jnp.dot(h[:, _HALF:], wa[...], preferred_element_type=_f32) + ba[...]
    t = jnp.tanh(jnp.dot(xn + xa, wt[...], preferred_element_type=_f32) + bt[...])
    a = jax.nn.sigmoid(jnp.sum(t * va[...], axis=1, keepdims=True))
    m = a * xn + (1.0 - a) * xa
    dis = lax.rsqrt(deg_ref[...] + 1.0)
    p = dis * m
    p0_ref[...] = p[:, :_HD]
    p1_ref[...] = p[:, _HD:]


@jax.jit
def _dense1_call(x, deg, wi, bi, wn, bn, wa, ba, wt, bt, va):
    full = lambda s: pl.BlockSpec(s, lambda i: (0,) * len(s))
    return pl.pallas_call(
        _dense1_body,
        grid=(_NBLK,),
        in_specs=[
            pl.BlockSpec((_RB, _D), lambda i: (i, 0)),
            pl.BlockSpec((_RB, 1), lambda i: (i, 0)),
            full((_D, _H)), full((1, _H)),
            full((_HALF, _H)), full((1, _H)),
            full((_HALF, _H)), full((1, _H)),
            full((_H, _H)), full((1, _H)),
            full((1, _H)),
        ],
        out_specs=[pl.BlockSpec((_RB, _HD), lambda i: (i, 0)),
                   pl.BlockSpec((_RB, _HD), lambda i: (i, 0))],
        out_shape=[jax.ShapeDtypeStruct((_NPAD, _HD), _f32),
                   jax.ShapeDtypeStruct((_NPAD, _HD), _f32)],
    )(x, deg, wi, bi, wn, bn, wa, ba, wt, bt, va)


def _dense2_body(q_ref, p0_ref, p1_ref, deg_ref, wu, bu, wc, bc, o_ref):
    q0 = q_ref[0] + p0_ref[...]
    q1 = q_ref[1] + p1_ref[...]
    aggr = lax.rsqrt(deg_ref[...] + 1.0) * jnp.concatenate([q0, q1], axis=-1)
    h2 = _leaky(jnp.dot(aggr, wu[...], preferred_element_type=_f32) + bu[...])
    o_ref[...] = jnp.dot(h2, wc[...], preferred_element_type=_f32) + bc[...]


@jax.jit
def _dense2_call(q, p0, p1, deg, wu, bu, wc, bc):
    full = lambda s: pl.BlockSpec(s, lambda i: (0,) * len(s))
    return pl.pallas_call(
        _dense2_body,
        grid=(_NBLK,),
        in_specs=[
            pl.BlockSpec((_NCORE, _RB, _HD), lambda i: (0, i, 0)),
            pl.BlockSpec((_RB, _HD), lambda i: (i, 0)),
            pl.BlockSpec((_RB, _HD), lambda i: (i, 0)),
            pl.BlockSpec((_RB, 1), lambda i: (i, 0)),
            full((_H, _H)), full((1, _H)),
            full((_H, _OUT)), full((1, _OUT)),
        ],
        out_specs=pl.BlockSpec((_RB, _OUT), lambda i: (i, 0)),
        out_shape=jax.ShapeDtypeStruct((_N, _OUT), _f32),
    )(q, p0, p1, deg, wu, bu, wc, bc)


def kernel(x, edge_index, W_in, b_in, W_nor, b_nor, W_abnor, b_abnor,
           W_att, b_att, v_att, W_upd, b_upd, W_cls, b_cls):
    ei3 = edge_index.reshape(2, _E // _CH, _CH)
    pk2, col2 = _pack_call(ei3)
    deg2 = _hist_call(col2)
    deg = (deg2[0] + deg2[1]).reshape(_NPAD, 1)
    p0, p1 = _dense1_call(x, deg, W_in, b_in.reshape(1, -1), W_nor,
                          b_nor.reshape(1, -1), W_abnor, b_abnor.reshape(1, -1),
                          W_att, b_att.reshape(1, -1), v_att.reshape(1, -1))
    q = _agg_call(p0, p1, pk2)
    return _dense2_call(q, p0, p1, deg, W_upd, b_upd.reshape(1, -1),
                        W_cls, b_cls.reshape(1, -1))


# speedup vs baseline: 1.0103x; 1.0103x over previous
import functools

import jax
import jax.numpy as jnp
from jax import lax
from jax.experimental import pallas as pl
from jax.experimental.pallas import tpu as pltpu
from jax.experimental.pallas import tpu_sc as plsc

_N = 10000
_D = 128
_H = 128
_HALF = 64
_OUT = 2
_NEG = 0.01

_NPAD = 10112
_RB = 632
_NBLK = _NPAD // _RB
_NCORE = 2
_NSUB = 16
_NW = _NCORE * _NSUB
_CH = 128
_CPW = 80
_NCHUNK = _NW * _CPW
_EPAD = _NCHUNK * _CH
_CPT = _NCHUNK // _NSUB
_HD = _D // 2

_f32 = jnp.float32
_E = 320000

_vmesh = plsc.VectorSubcoreMesh(core_axis_name="core", subcore_axis_name="subcore")
_sc_params = pltpu.CompilerParams(needs_layout_passes=False)


def _hist_body(col_hbm, out_hbm, col_v, hist_v, blk_v, acc_v, h_sh):
    cid = lax.axis_index("core")
    sid = lax.axis_index("subcore")
    w = cid * _NSUB + sid
    pltpu.sync_copy(col_hbm.at[pl.ds(w * _CPW, _CPW)], col_v)

    @pl.loop(0, _NPAD // 16)
    def _zero(i):
        hist_v[pl.ds(i * 16, 16)] = jnp.zeros((16,), _f32)

    ones = jnp.ones((16,), _f32)

    @pl.loop(0, _CPW)
    def _chunk(j):
        @pl.loop(0, _CH // 16)
        def _vec(c):
            iv = col_v[j, pl.ds(c * 16, 16)]
            plsc.addupdate_scatter(hist_v, [iv], ones)

    pltpu.sync_copy(hist_v, h_sh.at[sid])
    plsc.subcore_barrier()

    _HB = _NPAD // _CH
    _BPS = 5

    @pl.loop(0, _BPS)
    def _rb(t):
        blk = sid * _BPS + t

        @pl.when(blk < _HB)
        def _one():
            off = pl.multiple_of(blk * _CH, _CH)
            pltpu.sync_copy(h_sh.at[:, pl.ds(off, _CH)], blk_v)

            @pl.loop(0, _CH // 16)
            def _acc(i):
                s = blk_v[0, pl.ds(i * 16, 16)]
                for j in range(1, _NSUB):
                    s = s + blk_v[j, pl.ds(i * 16, 16)]
                acc_v[pl.ds(i * 16, 16)] = s

            pltpu.sync_copy(acc_v, out_hbm.at[cid, pl.ds(off, _CH)])


@jax.jit
def _hist_call(col2):
    return pl.kernel(
        _hist_body,
        out_type=jax.ShapeDtypeStruct((_NCORE, _NPAD), _f32),
        mesh=_vmesh,
        compiler_params=_sc_params,
        scratch_types=[
            pltpu.VMEM((_CPW, _CH), jnp.int32),
            pltpu.VMEM((_NPAD,), _f32),
            pltpu.VMEM((_NSUB, _CH), _f32),
            pltpu.VMEM((_CH,), _f32),
            pltpu.VMEM_SHARED((_NSUB, _NPAD), _f32),
        ],
    )(col2)


_NBUF = 4
_MASK = (1 << 14) - 1


def _agg_body(p0_hbm, p1_hbm, pk_hbm, q_hbm, pk_v, ridx_r, cidx_r, rows_v,
              p_sh, q_sh, *sems):
    cid = lax.axis_index("core")
    sid = lax.axis_index("subcore")

    @pl.when(cid == 0)
    def _s0():
        pltpu.sync_copy(p0_hbm.at[pl.ds(sid * _RB, _RB)],
                        p_sh.at[pl.ds(sid * _RB, _RB)])

    @pl.when(cid == 1)
    def _s1():
        pltpu.sync_copy(p1_hbm.at[pl.ds(sid * _RB, _RB)],
                        p_sh.at[pl.ds(sid * _RB, _RB)])

    @pl.loop(0, _CH)
    def _zr(i):
        @pl.loop(0, _HD // 16)
        def _zc(c):
            rows_v[i, pl.ds(c * 16, 16)] = jnp.zeros((16,), _f32)

    @pl.loop(0, _RB // _CH)
    def _zs(k):
        pltpu.sync_copy(rows_v.at[pl.ds(0, _CH)],
                        q_sh.at[pl.ds(sid * _RB + k * _CH, _CH)])

    _tail = _RB - (_RB // _CH) * _CH
    pltpu.sync_copy(rows_v.at[pl.ds(0, _tail)],
                    q_sh.at[pl.ds(sid * _RB + (_RB // _CH) * _CH, _tail)])

    _CHF = _CPT // 2

    pltpu.sync_copy(pk_hbm.at[pl.ds(sid * _CPT, _CHF)], pk_v)

    plsc.subcore_barrier()

    def _buf(b):
        return rows_v.at[pl.ds(b * _CH, _CH)]

    def _unpack(c, b):
        for k in range(_CH // 16):
            pk = pk_v[c, pl.ds(k * 16, 16)]
            ridx_r[b, pl.ds(k * 16, 16)] = pk & _MASK
            cidx_r[b, pl.ds(k * 16, 16)] = pk >> 14

    def _gather(b):
        pltpu.async_copy(p_sh.at[ridx_r.at[b]], _buf(b), sems[b])

    def _half(h):
        for b in range(_NBUF):
            _unpack(b, b)
            _gather(b)

        @pl.loop(0, _CHF // _NBUF)
        def _steady(s):
            for b in range(_NBUF):
                c = s * _NBUF + b
                pltpu.make_async_copy(p_sh.at[ridx_r.at[b]], _buf(b),
                                      sems[b]).wait()
                pltpu.sync_copy(_buf(b), q_sh.at[cidx_r.at[b]], add=True)
                _unpack(lax.rem(c + _NBUF, _CHF), b)
                _gather(b)

        for b in range(_NBUF):
            pltpu.make_async_copy(p_sh.at[ridx_r.at[b]], _buf(b),
                                  sems[b]).wait()

    _half(0)
    pltpu.sync_copy(pk_hbm.at[pl.ds(sid * _CPT + _CHF, _CHF)], pk_v)
    _half(1)

    plsc.subcore_barrier()

    pltpu.sync_copy(q_sh.at[pl.ds(sid * _RB, _RB)],
                    q_hbm.at[cid, pl.ds(sid * _RB, _RB)])


@jax.jit
def _agg_call(p0, p1, packed):
    return pl.kernel(
        _agg_body,
        out_type=jax.ShapeDtypeStruct((_NCORE, _NPAD, _HD), _f32),
        mesh=_vmesh,
        compiler_params=pltpu.CompilerParams(use_tc_tiling_on_sc=False),
        scratch_types=[
            pltpu.VMEM((_CPT // 2, _CH), jnp.int32),
            pltpu.VMEM((_NBUF, _CH), jnp.int32),
            pltpu.VMEM((_NBUF, _CH), jnp.int32),
            pltpu.VMEM((_NBUF * _CH, _HD), _f32),
            pltpu.VMEM_SHARED((_NPAD, _HD), _f32),
            pltpu.VMEM_SHARED((_NPAD, _HD), _f32),
        ] + [pltpu.SemaphoreType.DMA] * _NBUF,
    )(p0, p1, packed)


_EB = 8192
_EGRID = _EPAD // _EB


def _pack_body(ei_ref, pk_ref, col_ref):
    i = pl.program_id(0)
    base = i * _EB
    off = base + jax.lax.broadcasted_iota(jnp.int32, (_EB // _CH, _CH), 0) * _CH \
        + jax.lax.broadcasted_iota(jnp.int32, (_EB // _CH, _CH), 1)
    valid = off < _E
    row = jnp.where(valid, ei_ref[0].reshape(_EB // _CH, _CH), 0)
    col = jnp.where(valid, ei_ref[1].reshape(_EB // _CH, _CH), _NPAD - 1)
    pk_ref[...] = row | (col << 14)
    col_ref[...] = col


@jax.jit
def _pack_call(ei):
    return pl.pallas_call(
        _pack_body,
        grid=(_EGRID,),
        in_specs=[pl.BlockSpec((2, _EB), lambda i: (0, i))],
        out_specs=[pl.BlockSpec((_EB // _CH, _CH), lambda i: (i, 0)),
                   pl.BlockSpec((_EB // _CH, _CH), lambda i: (i, 0))],
        out_shape=[jax.ShapeDtypeStruct((_NCHUNK, _CH), jnp.int32),
                   jax.ShapeDtypeStruct((_NCHUNK, _CH), jnp.int32)],
    )(ei)


def _leaky(v):
    return jnp.where(v > 0, v, v * _NEG)


def _dense1_body(x_ref, deg_ref, wi, bi, wn, bn, wa, ba, wt, bt, va,
                 p0_ref, p1_ref):
    xb = x_ref[...]
    h = _leaky(jnp.dot(xb, wi[...], preferred_element_type=_f32) + bi[...])
    xn = jnp.dot(h[:, :_HALF], wn[...], preferred_element_type=_f32) + bn[...]
    xa = jnp.dot(h[:, _HALF:], wa[...], preferred_element_type=_f32) + ba[...]
    t = jnp.tanh(jnp.dot(xn + xa, wt[...], preferred_element_type=_f32) + bt[...])
    a = jax.nn.sigmoid(jnp.sum(t * va[...], axis=1, keepdims=True))
    m = a * xn + (1.0 - a) * xa
    dis = lax.rsqrt(deg_ref[...] + 1.0)
    p = dis * m
    p0_ref[...] = p[:, :_HD]
    p1_ref[...] = p[:, _HD:]


@jax.jit
def _dense1_call(x, deg, wi, bi, wn, bn, wa, ba, wt, bt, va):
    full = lambda s: pl.BlockSpec(s, lambda i: (0,) * len(s))
    return pl.pallas_call(
        _dense1_body,
        grid=(_NBLK,),
        in_specs=[
            pl.BlockSpec((_RB, _D), lambda i: (i, 0)),
            pl.BlockSpec((_RB, 1), lambda i: (i, 0)),
            full((_D, _H)), full((1, _H)),
            full((_HALF, _H)), full((1, _H)),
            full((_HALF, _H)), full((1, _H)),
            full((_H, _H)), full((1, _H)),
            full((1, _H)),
        ],
        out_specs=[pl.BlockSpec((_RB, _HD), lambda i: (i, 0)),
                   pl.BlockSpec((_RB, _HD), lambda i: (i, 0))],
        out_shape=[jax.ShapeDtypeStruct((_NPAD, _HD), _f32),
                   jax.ShapeDtypeStruct((_NPAD, _HD), _f32)],
    )(x, deg, wi, bi, wn, bn, wa, ba, wt, bt, va)


def _dense2_body(q_ref, p0_ref, p1_ref, deg_ref, wu, bu, wc, bc, o_ref):
    q0 = q_ref[0] + p0_ref[...]
    q1 = q_ref[1] + p1_ref[...]
    aggr = lax.rsqrt(deg_ref[...] + 1.0) * jnp.concatenate([q0, q1], axis=-1)
    h2 = _leaky(jnp.dot(aggr, wu[...], preferred_element_type=_f32) + bu[...])
    o_ref[...] = jnp.dot(h2, wc[...], preferred_element_type=_f32) + bc[...]


@jax.jit
def _dense2_call(q, p0, p1, deg, wu, bu, wc, bc):
    full = lambda s: pl.BlockSpec(s, lambda i: (0,) * len(s))
    return pl.pallas_call(
        _dense2_body,
        grid=(_NBLK,),
        in_specs=[
            pl.BlockSpec((_NCORE, _RB, _HD), lambda i: (0, i, 0)),
            pl.BlockSpec((_RB, _HD), lambda i: (i, 0)),
            pl.BlockSpec((_RB, _HD), lambda i: (i, 0)),
            pl.BlockSpec((_RB, 1), lambda i: (i, 0)),
            full((_H, _H)), full((1, _H)),
            full((_H, _OUT)), full((1, _OUT)),
        ],
        out_specs=pl.BlockSpec((_RB, _OUT), lambda i: (i, 0)),
        out_shape=jax.ShapeDtypeStruct((_N, _OUT), _f32),
    )(q, p0, p1, deg, wu, bu, wc, bc)


def kernel(x, edge_index, W_in, b_in, W_nor, b_nor, W_abnor, b_abnor,
           W_att, b_att, v_att, W_upd, b_upd, W_cls, b_cls):
    pk2, col2 = _pack_call(edge_index)
    deg2 = _hist_call(col2)
    deg = (deg2[0] + deg2[1]).reshape(_NPAD, 1)
    p0, p1 = _dense1_call(x, deg, W_in, b_in.reshape(1, -1), W_nor,
                          b_nor.reshape(1, -1), W_abnor, b_abnor.reshape(1, -1),
                          W_att, b_att.reshape(1, -1), v_att.reshape(1, -1))
    q = _agg_call(p0, p1, pk2)
    return _dense2_call(q, p0, p1, deg, W_upd, b_upd.reshape(1, -1),
                        W_cls, b_cls.reshape(1, -1))


# speedup vs baseline: 1.0217x; 1.0113x over previous
import functools

import jax
import jax.numpy as jnp
from jax import lax
from jax.experimental import pallas as pl
from jax.experimental.pallas import tpu as pltpu
from jax.experimental.pallas import tpu_sc as plsc

_N = 10000
_D = 128
_H = 128
_HALF = 64
_OUT = 2
_NEG = 0.01

_NPAD = 10112
_RB = 632
_NBLK = _NPAD // _RB
_NCORE = 2
_NSUB = 16
_NW = _NCORE * _NSUB
_CH = 128
_CPW = 80
_NCHUNK = _NW * _CPW
_EPAD = _NCHUNK * _CH
_CPT = _NCHUNK // _NSUB
_HD = _D // 2

_f32 = jnp.float32
_E = 320000

_vmesh = plsc.VectorSubcoreMesh(core_axis_name="core", subcore_axis_name="subcore")
_sc_params = pltpu.CompilerParams(needs_layout_passes=False)


def _hist_body(col_hbm, out_hbm, col_v, hist_v, blk_v, acc_v, h_sh):
    cid = lax.axis_index("core")
    sid = lax.axis_index("subcore")
    w = cid * _NSUB + sid
    pltpu.sync_copy(col_hbm.at[pl.ds(w * _CPW, _CPW)], col_v)

    @pl.loop(0, _NPAD // 16)
    def _zero(i):
        hist_v[pl.ds(i * 16, 16)] = jnp.zeros((16,), _f32)

    ones = jnp.ones((16,), _f32)

    @pl.loop(0, _CPW)
    def _chunk(j):
        @pl.loop(0, _CH // 16)
        def _vec(c):
            iv = col_v[j, pl.ds(c * 16, 16)]
            plsc.addupdate_scatter(hist_v, [iv], ones)

    pltpu.sync_copy(hist_v, h_sh.at[sid])
    plsc.subcore_barrier()

    _HB = _NPAD // _CH
    _BPS = 5

    @pl.loop(0, _BPS)
    def _rb(t):
        blk = sid * _BPS + t

        @pl.when(blk < _HB)
        def _one():
            off = pl.multiple_of(blk * _CH, _CH)
            pltpu.sync_copy(h_sh.at[:, pl.ds(off, _CH)], blk_v)

            @pl.loop(0, _CH // 16)
            def _acc(i):
                s = blk_v[0, pl.ds(i * 16, 16)]
                for j in range(1, _NSUB):
                    s = s + blk_v[j, pl.ds(i * 16, 16)]
                acc_v[pl.ds(i * 16, 16)] = s

            pltpu.sync_copy(acc_v, out_hbm.at[cid, pl.ds(off, _CH)])


@jax.jit
def _hist_call(col2):
    return pl.kernel(
        _hist_body,
        out_type=jax.ShapeDtypeStruct((_NCORE, _NPAD), _f32),
        mesh=_vmesh,
        compiler_params=_sc_params,
        scratch_types=[
            pltpu.VMEM((_CPW, _CH), jnp.int32),
            pltpu.VMEM((_NPAD,), _f32),
            pltpu.VMEM((_NSUB, _CH), _f32),
            pltpu.VMEM((_CH,), _f32),
            pltpu.VMEM_SHARED((_NSUB, _NPAD), _f32),
        ],
    )(col2)


_NBUF = 2
_MASK = (1 << 14) - 1


def _agg_body(p0_hbm, p1_hbm, pk_hbm, q_hbm, pk_v, ridx_r, cidx_r, rows_v,
              p_sh, q_sh, *sems):
    cid = lax.axis_index("core")
    sid = lax.axis_index("subcore")

    @pl.when(cid == 0)
    def _s0():
        pltpu.sync_copy(p0_hbm.at[pl.ds(sid * _RB, _RB)],
                        p_sh.at[pl.ds(sid * _RB, _RB)])

    @pl.when(cid == 1)
    def _s1():
        pltpu.sync_copy(p1_hbm.at[pl.ds(sid * _RB, _RB)],
                        p_sh.at[pl.ds(sid * _RB, _RB)])

    @pl.loop(0, _CH)
    def _zr(i):
        @pl.loop(0, _HD // 16)
        def _zc(c):
            rows_v[i, pl.ds(c * 16, 16)] = jnp.zeros((16,), _f32)

    @pl.loop(0, _RB // _CH)
    def _zs(k):
        pltpu.sync_copy(rows_v.at[pl.ds(0, _CH)],
                        q_sh.at[pl.ds(sid * _RB + k * _CH, _CH)])

    _tail = _RB - (_RB // _CH) * _CH
    pltpu.sync_copy(rows_v.at[pl.ds(0, _tail)],
                    q_sh.at[pl.ds(sid * _RB + (_RB // _CH) * _CH, _tail)])

    pltpu.sync_copy(pk_hbm.at[pl.ds(sid * _CPT, _CPT)], pk_v)

    plsc.subcore_barrier()

    def _buf(b):
        return rows_v.at[pl.ds(b * _CH, _CH)]

    def _unpack(c, b):
        for k in range(_CH // 16):
            pk = pk_v[c, pl.ds(k * 16, 16)]
            ridx_r[b, pl.ds(k * 16, 16)] = pk & _MASK
            cidx_r[b, pl.ds(k * 16, 16)] = pk >> 14

    def _gather(b):
        pltpu.async_copy(p_sh.at[ridx_r.at[b]], _buf(b), sems[b])

    for b in range(_NBUF):
        _unpack(b, b)
        _gather(b)

    @pl.loop(0, _CPT // _NBUF)
    def _steady(s):
        for b in range(_NBUF):
            c = s * _NBUF + b
            pltpu.make_async_copy(p_sh.at[ridx_r.at[b]], _buf(b),
                                  sems[b]).wait()
            pltpu.sync_copy(_buf(b), q_sh.at[cidx_r.at[b]], add=True)
            _unpack(lax.rem(c + _NBUF, _CPT), b)
            _gather(b)

    for b in range(_NBUF):
        pltpu.make_async_copy(p_sh.at[ridx_r.at[b]], _buf(b), sems[b]).wait()

    plsc.subcore_barrier()

    pltpu.sync_copy(q_sh.at[pl.ds(sid * _RB, _RB)],
                    q_hbm.at[cid, pl.ds(sid * _RB, _RB)])


@jax.jit
def _agg_call(p0, p1, packed):
    return pl.kernel(
        _agg_body,
        out_type=jax.ShapeDtypeStruct((_NCORE, _NPAD, _HD), _f32),
        mesh=_vmesh,
        compiler_params=pltpu.CompilerParams(use_tc_tiling_on_sc=False),
        scratch_types=[
            pltpu.VMEM((_CPT, _CH), jnp.int32),
            pltpu.VMEM((_NBUF, _CH), jnp.int32),
            pltpu.VMEM((_NBUF, _CH), jnp.int32),
            pltpu.VMEM((_NBUF * _CH, _HD), _f32),
            pltpu.VMEM_SHARED((_NPAD, _HD), _f32),
            pltpu.VMEM_SHARED((_NPAD, _HD), _f32),
        ] + [pltpu.SemaphoreType.DMA] * _NBUF,
    )(p0, p1, packed)


_EB = 8192
_EGRID = _EPAD // _EB


def _pack_body(ei_ref, pk_ref, col_ref):
    i = pl.program_id(0)
    base = i * _EB
    off = base + jax.lax.broadcasted_iota(jnp.int32, (_EB // _CH, _CH), 0) * _CH \
        + jax.lax.broadcasted_iota(jnp.int32, (_EB // _CH, _CH), 1)
    valid = off < _E
    row = jnp.where(valid, ei_ref[0].reshape(_EB // _CH, _CH), 0)
    col = jnp.where(valid, ei_ref[1].reshape(_EB // _CH, _CH), _NPAD - 1)
    pk_ref[...] = row | (col << 14)
    col_ref[...] = col


@jax.jit
def _pack_call(ei):
    return pl.pallas_call(
        _pack_body,
        grid=(_EGRID,),
        in_specs=[pl.BlockSpec((2, _EB), lambda i: (0, i))],
        out_specs=[pl.BlockSpec((_EB // _CH, _CH), lambda i: (i, 0)),
                   pl.BlockSpec((_EB // _CH, _CH), lambda i: (i, 0))],
        out_shape=[jax.ShapeDtypeStruct((_NCHUNK, _CH), jnp.int32),
                   jax.ShapeDtypeStruct((_NCHUNK, _CH), jnp.int32)],
    )(ei)


def _leaky(v):
    return jnp.where(v > 0, v, v * _NEG)


def _dense1_body(x_ref, deg_ref, wi, bi, wn, bn, wa, ba, wt, bt, va,
                 p0_ref, p1_ref):
    xb = x_ref[...]
    h = _leaky(jnp.dot(xb, wi[...], preferred_element_type=_f32) + bi[...])
    xn = jnp.dot(h[:, :_HALF], wn[...], preferred_element_type=_f32) + bn[...]
    xa = jnp.dot(h[:, _HALF:], wa[...], preferred_element_type=_f32) + ba[...]
    t = jnp.tanh(jnp.dot(xn + xa, wt[...], preferred_element_type=_f32) + bt[...])
    a = jax.nn.sigmoid(jnp.sum(t * va[...], axis=1, keepdims=True))
    m = a * xn + (1.0 - a) * xa
    dis = lax.rsqrt(deg_ref[...] + 1.0)
    p = dis * m
    p0_ref[...] = p[:, :_HD]
    p1_ref[...] = p[:, _HD:]


@jax.jit
def _dense1_call(x, deg, wi, bi, wn, bn, wa, ba, wt, bt, va):
    full = lambda s: pl.BlockSpec(s, lambda i: (0,) * len(s))
    return pl.pallas_call(
        _dense1_body,
        grid=(_NBLK,),
        in_specs=[
            pl.BlockSpec((_RB, _D), lambda i: (i, 0)),
            pl.BlockSpec((_RB, 1), lambda i: (i, 0)),
            full((_D, _H)), full((1, _H)),
            full((_HALF, _H)), full((1, _H)),
            full((_HALF, _H)), full((1, _H)),
            full((_H, _H)), full((1, _H)),
            full((1, _H)),
        ],
        out_specs=[pl.BlockSpec((_RB, _HD), lambda i: (i, 0)),
                   pl.BlockSpec((_RB, _HD), lambda i: (i, 0))],
        out_shape=[jax.ShapeDtypeStruct((_NPAD, _HD), _f32),
                   jax.ShapeDtypeStruct((_NPAD, _HD), _f32)],
    )(x, deg, wi, bi, wn, bn, wa, ba, wt, bt, va)


def _dense2_body(q_ref, p0_ref, p1_ref, deg_ref, wu, bu, wc, bc, o_ref):
    q0 = q_ref[0] + p0_ref[...]
    q1 = q_ref[1] + p1_ref[...]
    aggr = lax.rsqrt(deg_ref[...] + 1.0) * jnp.concatenate([q0, q1], axis=-1)
    h2 = _leaky(jnp.dot(aggr, wu[...], preferred_element_type=_f32) + bu[...])
    o_ref[...] = jnp.dot(h2, wc[...], preferred_element_type=_f32) + bc[...]


@jax.jit
def _dense2_call(q, p0, p1, deg, wu, bu, wc, bc):
    full = lambda s: pl.BlockSpec(s, lambda i: (0,) * len(s))
    return pl.pallas_call(
        _dense2_body,
        grid=(_NBLK,),
        in_specs=[
            pl.BlockSpec((_NCORE, _RB, _HD), lambda i: (0, i, 0)),
            pl.BlockSpec((_RB, _HD), lambda i: (i, 0)),
            pl.BlockSpec((_RB, _HD), lambda i: (i, 0)),
            pl.BlockSpec((_RB, 1), lambda i: (i, 0)),
            full((_H, _H)), full((1, _H)),
            full((_H, _OUT)), full((1, _OUT)),
        ],
        out_specs=pl.BlockSpec((_RB, _OUT), lambda i: (i, 0)),
        out_shape=jax.ShapeDtypeStruct((_N, _OUT), _f32),
    )(q, p0, p1, deg, wu, bu, wc, bc)


def kernel(x, edge_index, W_in, b_in, W_nor, b_nor, W_abnor, b_abnor,
           W_att, b_att, v_att, W_upd, b_upd, W_cls, b_cls):
    pk2, col2 = _pack_call(edge_index)
    deg2 = _hist_call(col2)
    deg = (deg2[0] + deg2[1]).reshape(_NPAD, 1)
    p0, p1 = _dense1_call(x, deg, W_in, b_in.reshape(1, -1), W_nor,
                          b_nor.reshape(1, -1), W_abnor, b_abnor.reshape(1, -1),
                          W_att, b_att.reshape(1, -1), v_att.reshape(1, -1))
    q = _agg_call(p0, p1, pk2)
    return _dense2_call(q, p0, p1, deg, W_upd, b_upd.reshape(1, -1),
                        W_cls, b_cls.reshape(1, -1))


# speedup vs baseline: 1.0226x; 1.0009x over previous
import jax
import jax.numpy as jnp
from jax import lax
from jax.experimental import pallas as pl
from jax.experimental.pallas import tpu as pltpu
from jax.experimental.pallas import tpu_sc as plsc

_N = 10000
_D = 128
_H = 128
_HALF = 64
_OUT = 2
_NEG = 0.01

_NPAD = 10112
_RB = 632
_NBLK = _NPAD // _RB
_NCORE = 2
_NSUB = 16
_NW = _NCORE * _NSUB
_CH = 128
_CPW = 80
_NCHUNK = _NW * _CPW
_EPAD = _NCHUNK * _CH
_CPT = _NCHUNK // _NSUB
_HD = _D // 2

_f32 = jnp.float32
_E = 320000

_vmesh = plsc.VectorSubcoreMesh(core_axis_name="core", subcore_axis_name="subcore")
_sc_params = pltpu.CompilerParams(needs_layout_passes=False)


def _hist_body(col_hbm, out_hbm, col_v, hist_v, blk_v, acc_v, h_sh):
    cid = lax.axis_index("core")
    sid = lax.axis_index("subcore")
    w = cid * _NSUB + sid
    pltpu.sync_copy(col_hbm.at[pl.ds(w * _CPW, _CPW)], col_v)

    @pl.loop(0, _NPAD // 16)
    def _zero(i):
        hist_v[pl.ds(i * 16, 16)] = jnp.zeros((16,), _f32)

    ones = jnp.ones((16,), _f32)

    @pl.loop(0, _CPW)
    def _chunk(j):
        @pl.loop(0, _CH // 16)
        def _vec(c):
            iv = col_v[j, pl.ds(c * 16, 16)]
            plsc.addupdate_scatter(hist_v, [iv], ones)

    pltpu.sync_copy(hist_v, h_sh.at[sid])
    plsc.subcore_barrier()

    _HB = _NPAD // _CH
    _BPS = 5

    @pl.loop(0, _BPS)
    def _rb(t):
        blk = sid * _BPS + t

        @pl.when(blk < _HB)
        def _one():
            off = pl.multiple_of(blk * _CH, _CH)
            pltpu.sync_copy(h_sh.at[:, pl.ds(off, _CH)], blk_v)

            @pl.loop(0, _CH // 16)
            def _acc(i):
                s = blk_v[0, pl.ds(i * 16, 16)]
                for j in range(1, _NSUB):
                    s = s + blk_v[j, pl.ds(i * 16, 16)]
                acc_v[pl.ds(i * 16, 16)] = s

            pltpu.sync_copy(acc_v, out_hbm.at[cid, pl.ds(off, _CH)])


@jax.jit
def _hist_call(col2):
    return pl.kernel(
        _hist_body,
        out_type=jax.ShapeDtypeStruct((_NCORE, _NPAD), _f32),
        mesh=_vmesh,
        compiler_params=_sc_params,
        scratch_types=[
            pltpu.VMEM((_CPW, _CH), jnp.int32),
            pltpu.VMEM((_NPAD,), _f32),
            pltpu.VMEM((_NSUB, _CH), _f32),
            pltpu.VMEM((_CH,), _f32),
            pltpu.VMEM_SHARED((_NSUB, _NPAD), _f32),
        ],
    )(col2)


_NBUF = 2
_MASK = (1 << 14) - 1


def _agg_body(p0_hbm, p1_hbm, pk_hbm, q_hbm, pk_v, ridx_r, cidx_r, rows_v,
              p_sh, q_sh, *sems):
    cid = lax.axis_index("core")
    sid = lax.axis_index("subcore")

    @pl.when(cid == 0)
    def _s0():
        pltpu.sync_copy(p0_hbm.at[pl.ds(sid * _RB, _RB)],
                        p_sh.at[pl.ds(sid * _RB, _RB)])

    @pl.when(cid == 1)
    def _s1():
        pltpu.sync_copy(p1_hbm.at[pl.ds(sid * _RB, _RB)],
                        p_sh.at[pl.ds(sid * _RB, _RB)])

    @pl.loop(0, _CH)
    def _zr(i):
        @pl.loop(0, _HD // 16)
        def _zc(c):
            rows_v[i, pl.ds(c * 16, 16)] = jnp.zeros((16,), _f32)

    @pl.loop(0, _RB // _CH)
    def _zs(k):
        pltpu.sync_copy(rows_v.at[pl.ds(0, _CH)],
                        q_sh.at[pl.ds(sid * _RB + k * _CH, _CH)])

    _tail = _RB - (_RB // _CH) * _CH
    pltpu.sync_copy(rows_v.at[pl.ds(0, _tail)],
                    q_sh.at[pl.ds(sid * _RB + (_RB // _CH) * _CH, _tail)])

    pltpu.sync_copy(pk_hbm.at[pl.ds(sid * _CPT, _CPT)], pk_v)

    plsc.subcore_barrier()

    def _buf(b):
        return rows_v.at[pl.ds(b * _CH, _CH)]

    def _unpack(c, b):
        for k in range(_CH // 16):
            pk = pk_v[c, pl.ds(k * 16, 16)]
            ridx_r[b, pl.ds(k * 16, 16)] = pk & _MASK
            cidx_r[b, pl.ds(k * 16, 16)] = pk >> 14

    def _gather(b):
        pltpu.async_copy(p_sh.at[ridx_r.at[b]], _buf(b), sems[b])

    for b in range(_NBUF):
        _unpack(b, b)
        _gather(b)

    @pl.loop(0, _CPT // _NBUF)
    def _steady(s):
        for b in range(_NBUF):
            c = s * _NBUF + b
            pltpu.make_async_copy(p_sh.at[ridx_r.at[b]], _buf(b),
                                  sems[b]).wait()
            pltpu.sync_copy(_buf(b), q_sh.at[cidx_r.at[b]], add=True)
            _unpack(lax.rem(c + _NBUF, _CPT), b)
            _gather(b)

    for b in range(_NBUF):
        pltpu.make_async_copy(p_sh.at[ridx_r.at[b]], _buf(b), sems[b]).wait()

    plsc.subcore_barrier()

    pltpu.sync_copy(q_sh.at[pl.ds(sid * _RB, _RB)],
                    q_hbm.at[cid, pl.ds(sid * _RB, _RB)])


@jax.jit
def _agg_call(p0, p1, packed):
    return pl.kernel(
        _agg_body,
        out_type=jax.ShapeDtypeStruct((_NCORE, _NPAD, _HD), _f32),
        mesh=_vmesh,
        compiler_params=pltpu.CompilerParams(use_tc_tiling_on_sc=False),
        scratch_types=[
            pltpu.VMEM((_CPT, _CH), jnp.int32),
            pltpu.VMEM((_NBUF, _CH), jnp.int32),
            pltpu.VMEM((_NBUF, _CH), jnp.int32),
            pltpu.VMEM((_NBUF * _CH, _HD), _f32),
            pltpu.VMEM_SHARED((_NPAD, _HD), _f32),
            pltpu.VMEM_SHARED((_NPAD, _HD), _f32),
        ] + [pltpu.SemaphoreType.DMA] * _NBUF,
    )(p0, p1, packed)


_EB = 8192
_EGRID = _EPAD // _EB


def _pack_body(ei_ref, pk_ref, col_ref):
    i = pl.program_id(0)
    base = i * _EB
    off = base + jax.lax.broadcasted_iota(jnp.int32, (_EB // _CH, _CH), 0) * _CH \
        + jax.lax.broadcasted_iota(jnp.int32, (_EB // _CH, _CH), 1)
    valid = off < _E
    row = jnp.where(valid, ei_ref[0].reshape(_EB // _CH, _CH), 0)
    col = jnp.where(valid, ei_ref[1].reshape(_EB // _CH, _CH), _NPAD - 1)
    pk_ref[...] = row | (col << 14)
    col_ref[...] = col


@jax.jit
def _pack_call(ei):
    return pl.pallas_call(
        _pack_body,
        grid=(_EGRID,),
        in_specs=[pl.BlockSpec((2, _EB), lambda i: (0, i))],
        out_specs=[pl.BlockSpec((_EB // _CH, _CH), lambda i: (i, 0)),
                   pl.BlockSpec((_EB // _CH, _CH), lambda i: (i, 0))],
        out_shape=[jax.ShapeDtypeStruct((_NCHUNK, _CH), jnp.int32),
                   jax.ShapeDtypeStruct((_NCHUNK, _CH), jnp.int32)],
    )(ei)


def _leaky(v):
    return jnp.where(v > 0, v, v * _NEG)


def _dense1_body(x_ref, deg_ref, wi, bi, wn, bn, wa, ba, wt, bt, va,
                 p0_ref, p1_ref):
    xb = x_ref[...]
    h = _leaky(jnp.dot(xb, wi[...], preferred_element_type=_f32) + bi[...])
    xn = jnp.dot(h[:, :_HALF], wn[...], preferred_element_type=_f32) + bn[...]
    xa = jnp.dot(h[:, _HALF:], wa[...], preferred_element_type=_f32) + ba[...]
    t = jnp.tanh(jnp.dot(xn + xa, wt[...], preferred_element_type=_f32) + bt[...])
    a = jax.nn.sigmoid(jnp.sum(t * va[...], axis=1, keepdims=True))
    m = a * xn + (1.0 - a) * xa
    dis = lax.rsqrt(deg_ref[...] + 1.0)
    p = dis * m
    p0_ref[...] = p[:, :_HD]
    p1_ref[...] = p[:, _HD:]


@jax.jit
def _dense1_call(x, deg, wi, bi, wn, bn, wa, ba, wt, bt, va):
    full = lambda s: pl.BlockSpec(s, lambda i: (0,) * len(s))
    return pl.pallas_call(
        _dense1_body,
        grid=(_NBLK,),
        in_specs=[
            pl.BlockSpec((_RB, _D), lambda i: (i, 0)),
            pl.BlockSpec((_RB, 1), lambda i: (i, 0)),
            full((_D, _H)), full((1, _H)),
            full((_HALF, _H)), full((1, _H)),
            full((_HALF, _H)), full((1, _H)),
            full((_H, _H)), full((1, _H)),
            full((1, _H)),
        ],
        out_specs=[pl.BlockSpec((_RB, _HD), lambda i: (i, 0)),
                   pl.BlockSpec((_RB, _HD), lambda i: (i, 0))],
        out_shape=[jax.ShapeDtypeStruct((_NPAD, _HD), _f32),
                   jax.ShapeDtypeStruct((_NPAD, _HD), _f32)],
    )(x, deg, wi, bi, wn, bn, wa, ba, wt, bt, va)


def _dense2_body(q_ref, p0_ref, p1_ref, deg_ref, wu, bu, wc, bc, o_ref):
    q0 = q_ref[0] + p0_ref[...]
    q1 = q_ref[1] + p1_ref[...]
    aggr = lax.rsqrt(deg_ref[...] + 1.0) * jnp.concatenate([q0, q1], axis=-1)
    h2 = _leaky(jnp.dot(aggr, wu[...], preferred_element_type=_f32) + bu[...])
    o_ref[...] = jnp.dot(h2, wc[...], preferred_element_type=_f32) + bc[...]


@jax.jit
def _dense2_call(q, p0, p1, deg, wu, bu, wc, bc):
    full = lambda s: pl.BlockSpec(s, lambda i: (0,) * len(s))
    return pl.pallas_call(
        _dense2_body,
        grid=(_NBLK,),
        in_specs=[
            pl.BlockSpec((_NCORE, _RB, _HD), lambda i: (0, i, 0)),
            pl.BlockSpec((_RB, _HD), lambda i: (i, 0)),
            pl.BlockSpec((_RB, _HD), lambda i: (i, 0)),
            pl.BlockSpec((_RB, 1), lambda i: (i, 0)),
            full((_H, _H)), full((1, _H)),
            full((_H, _OUT)), full((1, _OUT)),
        ],
        out_specs=pl.BlockSpec((_RB, _OUT), lambda i: (i, 0)),
        out_shape=jax.ShapeDtypeStruct((_N, _OUT), _f32),
    )(q, p0, p1, deg, wu, bu, wc, bc)


def kernel(x, edge_index, W_in, b_in, W_nor, b_nor, W_abnor, b_abnor,
           W_att, b_att, v_att, W_upd, b_upd, W_cls, b_cls):
    pk2, col2 = _pack_call(edge_index)
    deg2 = _hist_call(col2)
    deg = (deg2[0] + deg2[1]).reshape(_NPAD, 1)
    p0, p1 = _dense1_call(x, deg, W_in, b_in.reshape(1, -1), W_nor,
                          b_nor.reshape(1, -1), W_abnor, b_abnor.reshape(1, -1),
                          W_att, b_att.reshape(1, -1), v_att.reshape(1, -1))
    q = _agg_call(p0, p1, pk2)
    return _dense2_call(q, p0, p1, deg, W_upd, b_upd.reshape(1, -1),
                        W_cls, b_cls.reshape(1, -1))
